# Initial kernel scaffold; baseline (speedup 1.0000x reference)
import jax
import jax.numpy as jnp
from jax.experimental import pallas as pl


def kernel(X, adj_, W1, b1, W2, b2, W3, b3, W4, b4, W5, b5, W6, b6):
    raise NotImplementedError("write your pallas kernel here")


# speedup vs baseline: 1.0013x; 1.0013x over previous
import jax
import jax.numpy as jnp
from jax.experimental import pallas as pl
from jax.experimental.pallas import tpu as pltpu

_BM = 256


def _layer_kernel(a_ref, z_ref, w_ref, b_ref, o_ref, y_ref):
    @pl.when(pl.program_id(0) == 0)
    def _():
        y_ref[...] = jnp.dot(z_ref[...], w_ref[...],
                             preferred_element_type=jnp.float32)

    o_ref[...] = jnp.maximum(
        jnp.dot(a_ref[...], y_ref[...], preferred_element_type=jnp.float32)
        + b_ref[...],
        0.0,
    )


def _gcn_layer(adj, z, w, b):
    n, d_in = z.shape
    d_out = w.shape[1]
    return pl.pallas_call(
        _layer_kernel,
        grid=(pl.cdiv(n, _BM),),
        in_specs=[
            pl.BlockSpec((_BM, n), lambda i: (i, 0)),
            pl.BlockSpec((n, d_in), lambda i: (0, 0)),
            pl.BlockSpec((d_in, d_out), lambda i: (0, 0)),
            pl.BlockSpec((1, d_out), lambda i: (0, 0)),
        ],
        out_specs=pl.BlockSpec((_BM, d_out), lambda i: (i, 0)),
        out_shape=jax.ShapeDtypeStruct((n, d_out), jnp.float32),
        scratch_shapes=[pltpu.VMEM((n, d_out), jnp.float32)],
    )(adj, z, w, b.reshape(1, -1))


def kernel(X, adj_, W1, b1, W2, b2, W3, b3, W4, b4, W5, b5, W6, b6):
    z = X
    for w, b in ((W1, b1), (W2, b2), (W3, b3), (W4, b4), (W5, b5), (W6, b6)):
        z = _gcn_layer(adj_, z, w, b)
    return z


# speedup vs baseline: 1.1908x; 1.1892x over previous
import jax
import jax.numpy as jnp
from jax.experimental import pallas as pl
from jax.experimental.pallas import tpu as pltpu

_BM = 256


def _layer1_kernel(a_ref, z_ref, w_ref, b_ref, o_ref, abf_ref, y_ref):
    @pl.when(pl.program_id(0) == 0)
    def _():
        y_ref[...] = jnp.dot(
            z_ref[...], w_ref[...], preferred_element_type=jnp.float32
        ).astype(jnp.bfloat16)

    a = a_ref[...].astype(jnp.bfloat16)
    abf_ref[...] = a
    o_ref[...] = jnp.maximum(
        jnp.dot(a, y_ref[...], preferred_element_type=jnp.float32)
        + b_ref[...],
        0.0,
    )


def _layer_kernel(a_ref, z_ref, w_ref, b_ref, o_ref, y_ref):
    @pl.when(pl.program_id(0) == 0)
    def _():
        y_ref[...] = jnp.dot(
            z_ref[...], w_ref[...], preferred_element_type=jnp.float32
        ).astype(jnp.bfloat16)

    o_ref[...] = jnp.maximum(
        jnp.dot(a_ref[...], y_ref[...], preferred_element_type=jnp.float32)
        + b_ref[...],
        0.0,
    )


def _gcn_layer1(adj, z, w, b):
    n, d_in = z.shape
    d_out = w.shape[1]
    return pl.pallas_call(
        _layer1_kernel,
        grid=(pl.cdiv(n, _BM),),
        in_specs=[
            pl.BlockSpec((_BM, n), lambda i: (i, 0)),
            pl.BlockSpec((n, d_in), lambda i: (0, 0)),
            pl.BlockSpec((d_in, d_out), lambda i: (0, 0)),
            pl.BlockSpec((1, d_out), lambda i: (0, 0)),
        ],
        out_specs=(
            pl.BlockSpec((_BM, d_out), lambda i: (i, 0)),
            pl.BlockSpec((_BM, n), lambda i: (i, 0)),
        ),
        out_shape=(
            jax.ShapeDtypeStruct((n, d_out), jnp.float32),
            jax.ShapeDtypeStruct((n, n), jnp.bfloat16),
        ),
        scratch_shapes=[pltpu.VMEM((n, d_out), jnp.bfloat16)],
    )(adj, z, w, b.reshape(1, -1))


def _gcn_layer(adj_bf, z, w, b):
    n, d_in = z.shape
    d_out = w.shape[1]
    return pl.pallas_call(
        _layer_kernel,
        grid=(pl.cdiv(n, _BM),),
        in_specs=[
            pl.BlockSpec((_BM, n), lambda i: (i, 0)),
            pl.BlockSpec((n, d_in), lambda i: (0, 0)),
            pl.BlockSpec((d_in, d_out), lambda i: (0, 0)),
            pl.BlockSpec((1, d_out), lambda i: (0, 0)),
        ],
        out_specs=pl.BlockSpec((_BM, d_out), lambda i: (i, 0)),
        out_shape=jax.ShapeDtypeStruct((n, d_out), jnp.float32),
        scratch_shapes=[pltpu.VMEM((n, d_out), jnp.bfloat16)],
    )(adj_bf, z, w, b.reshape(1, -1))


def kernel(X, adj_, W1, b1, W2, b2, W3, b3, W4, b4, W5, b5, W6, b6):
    z, adj_bf = _gcn_layer1(adj_, X, W1, b1)
    for w, b in ((W2, b2), (W3, b3), (W4, b4), (W5, b5), (W6, b6)):
        z = _gcn_layer(adj_bf, z, w, b)
    return z


# speedup vs baseline: 1.3528x; 1.1360x over previous
import jax
import jax.numpy as jnp
from jax.experimental import pallas as pl
from jax.experimental.pallas import tpu as pltpu

_BM = 256
_BM2 = 512


def _layer1_kernel(a_ref, z_ref, w_ref, b_ref, o_ref, abf_ref, y_ref):
    @pl.when(pl.program_id(0) == 0)
    def _():
        y_ref[...] = jnp.dot(
            z_ref[...], w_ref[...], preferred_element_type=jnp.float32
        ).astype(jnp.bfloat16)

    a = a_ref[...].astype(jnp.bfloat16)
    abf_ref[...] = a
    o_ref[...] = jnp.maximum(
        jnp.dot(a, y_ref[...], preferred_element_type=jnp.float32)
        + b_ref[...],
        0.0,
    )


def _layer_kernel(a_ref, z_ref, w_ref, b_ref, o_ref, y_ref):
    @pl.when(pl.program_id(0) == 0)
    def _():
        y_ref[...] = jnp.dot(
            z_ref[...], w_ref[...], preferred_element_type=jnp.float32
        ).astype(jnp.bfloat16)

    o_ref[...] = jnp.maximum(
        jnp.dot(a_ref[...], y_ref[...], preferred_element_type=jnp.float32)
        + b_ref[...],
        0.0,
    )


def _gcn_layer1(adj, z, w, b):
    n, d_in = z.shape
    d_out = w.shape[1]
    return pl.pallas_call(
        _layer1_kernel,
        grid=(pl.cdiv(n, _BM),),
        in_specs=[
            pl.BlockSpec((_BM, n), lambda i: (i, 0)),
            pl.BlockSpec((n, d_in), lambda i: (0, 0)),
            pl.BlockSpec((d_in, d_out), lambda i: (0, 0)),
            pl.BlockSpec((1, d_out), lambda i: (0, 0)),
        ],
        out_specs=(
            pl.BlockSpec((_BM, d_out), lambda i: (i, 0)),
            pl.BlockSpec((_BM, n), lambda i: (i, 0)),
        ),
        out_shape=(
            jax.ShapeDtypeStruct((n, d_out), jnp.float32),
            jax.ShapeDtypeStruct((n, n), jnp.bfloat16),
        ),
        scratch_shapes=[pltpu.VMEM((n, d_out), jnp.bfloat16)],
    )(adj, z, w, b.reshape(1, -1))


def _gcn_layer(adj_bf, z, w, b):
    n, d_in = z.shape
    d_out = w.shape[1]
    return pl.pallas_call(
        _layer_kernel,
        grid=(pl.cdiv(n, _BM2),),
        in_specs=[
            pl.BlockSpec((_BM2, n), lambda i: (i, 0)),
            pl.BlockSpec((n, d_in), lambda i: (0, 0)),
            pl.BlockSpec((d_in, d_out), lambda i: (0, 0)),
            pl.BlockSpec((1, d_out), lambda i: (0, 0)),
        ],
        out_specs=pl.BlockSpec((_BM2, d_out), lambda i: (i, 0)),
        out_shape=jax.ShapeDtypeStruct((n, d_out), jnp.float32),
        scratch_shapes=[pltpu.VMEM((n, d_out), jnp.bfloat16)],
    )(adj_bf, z, w, b.reshape(1, -1))


def kernel(X, adj_, W1, b1, W2, b2, W3, b3, W4, b4, W5, b5, W6, b6):
    z, adj_bf = _gcn_layer1(adj_, X, W1, b1)
    for w, b in ((W2, b2), (W3, b3), (W4, b4), (W5, b5), (W6, b6)):
        z = _gcn_layer(adj_bf, z, w, b)
    return z


# speedup vs baseline: 1.3627x; 1.0073x over previous
import jax
import jax.numpy as jnp
from jax.experimental import pallas as pl
from jax.experimental.pallas import tpu as pltpu

_BM = 256
_BM2 = 1024


def _layer1_kernel(a_ref, z_ref, w_ref, b_ref, o_ref, abf_ref, y_ref):
    @pl.when(pl.program_id(0) == 0)
    def _():
        y_ref[...] = jnp.dot(
            z_ref[...], w_ref[...], preferred_element_type=jnp.float32
        ).astype(jnp.bfloat16)

    a = a_ref[...].astype(jnp.bfloat16)
    abf_ref[...] = a
    o_ref[...] = jnp.maximum(
        jnp.dot(a, y_ref[...], preferred_element_type=jnp.float32)
        + b_ref[...],
        0.0,
    )


def _layer_kernel(a_ref, z_ref, w_ref, b_ref, o_ref, y_ref):
    @pl.when(pl.program_id(0) == 0)
    def _():
        y_ref[...] = jnp.dot(
            z_ref[...], w_ref[...], preferred_element_type=jnp.float32
        ).astype(jnp.bfloat16)

    o_ref[...] = jnp.maximum(
        jnp.dot(a_ref[...], y_ref[...], preferred_element_type=jnp.float32)
        + b_ref[...],
        0.0,
    )


def _gcn_layer1(adj, z, w, b):
    n, d_in = z.shape
    d_out = w.shape[1]
    return pl.pallas_call(
        _layer1_kernel,
        grid=(pl.cdiv(n, _BM),),
        in_specs=[
            pl.BlockSpec((_BM, n), lambda i: (i, 0)),
            pl.BlockSpec((n, d_in), lambda i: (0, 0)),
            pl.BlockSpec((d_in, d_out), lambda i: (0, 0)),
            pl.BlockSpec((1, d_out), lambda i: (0, 0)),
        ],
        out_specs=(
            pl.BlockSpec((_BM, d_out), lambda i: (i, 0)),
            pl.BlockSpec((_BM, n), lambda i: (i, 0)),
        ),
        out_shape=(
            jax.ShapeDtypeStruct((n, d_out), jnp.float32),
            jax.ShapeDtypeStruct((n, n), jnp.bfloat16),
        ),
        scratch_shapes=[pltpu.VMEM((n, d_out), jnp.bfloat16)],
    )(adj, z, w, b.reshape(1, -1))


def _gcn_layer(adj_bf, z, w, b):
    n, d_in = z.shape
    d_out = w.shape[1]
    return pl.pallas_call(
        _layer_kernel,
        grid=(pl.cdiv(n, _BM2),),
        in_specs=[
            pl.BlockSpec((_BM2, n), lambda i: (i, 0)),
            pl.BlockSpec((n, d_in), lambda i: (0, 0)),
            pl.BlockSpec((d_in, d_out), lambda i: (0, 0)),
            pl.BlockSpec((1, d_out), lambda i: (0, 0)),
        ],
        out_specs=pl.BlockSpec((_BM2, d_out), lambda i: (i, 0)),
        out_shape=jax.ShapeDtypeStruct((n, d_out), jnp.float32),
        scratch_shapes=[pltpu.VMEM((n, d_out), jnp.bfloat16)],
    )(adj_bf, z, w, b.reshape(1, -1))


def kernel(X, adj_, W1, b1, W2, b2, W3, b3, W4, b4, W5, b5, W6, b6):
    z, adj_bf = _gcn_layer1(adj_, X, W1, b1)
    for w, b in ((W2, b2), (W3, b3), (W4, b4), (W5, b5), (W6, b6)):
        z = _gcn_layer(adj_bf, z, w, b)
    return z
